# baseline (speedup 1.0000x reference)
import functools

import jax
import jax.numpy as jnp
from jax import lax
from jax.experimental import pallas as pl
from jax.experimental.pallas import tpu as pltpu

T, D, H, E, A = 4096, 768, 64, 14, 128
TILE_T = 1024
K = 3


def _moe_kernel(x_ref, wg1_ref, bg1_ref, wg2_ref, bg2_ref, we_ref, be_ref,
                y_ref, comb_ref):
    e = pl.program_id(1)

    @pl.when(e == 0)
    def _gating():
        x = x_ref[...]
        h = jnp.maximum(
            jnp.dot(x, wg1_ref[...], preferred_element_type=jnp.float32)
            + bg1_ref[...], 0.0)
        logits_t = lax.dot_general(
            wg2_ref[...], h, (((0,), (1,)), ((), ())),
            preferred_element_type=jnp.float32) + bg2_ref[...].T
        m = jnp.max(logits_t, axis=0, keepdims=True)
        ex = jnp.exp(logits_t - m)
        w = ex / jnp.sum(ex, axis=0, keepdims=True)

        row = lax.broadcasted_iota(jnp.int32, w.shape, 0)
        mask = jnp.zeros(w.shape, jnp.bool_)
        for _ in range(K):
            cand = jnp.where(mask, -1.0, w)
            mx = jnp.max(cand, axis=0, keepdims=True)
            first = jnp.min(jnp.where(cand == mx, row, E), axis=0,
                            keepdims=True)
            mask = mask | (row == first)
        combine = jnp.where(mask, w, 0.0).T
        comb_ref[...] = combine
        y_ref[...] = jnp.dot(combine, be_ref[...],
                             preferred_element_type=jnp.float32)

    xe = jnp.dot(x_ref[...], we_ref[0], preferred_element_type=jnp.float32)
    onehot = (lax.broadcasted_iota(jnp.int32, (E, 1), 0) == e).astype(
        jnp.float32)
    ce = jnp.dot(comb_ref[...], onehot, preferred_element_type=jnp.float32)
    y_ref[...] += ce * xe


@functools.partial(jax.jit, static_argnames=())
def _run(x, Wg1, bg1, Wg2, bg2, We, be):
    grid = (T // TILE_T, E)
    return pl.pallas_call(
        _moe_kernel,
        grid=grid,
        in_specs=[
            pl.BlockSpec((TILE_T, D), lambda i, e: (i, 0)),
            pl.BlockSpec((D, H), lambda i, e: (0, 0)),
            pl.BlockSpec((1, H), lambda i, e: (0, 0)),
            pl.BlockSpec((H, E), lambda i, e: (0, 0)),
            pl.BlockSpec((1, E), lambda i, e: (0, 0)),
            pl.BlockSpec((1, D, A), lambda i, e: (e, 0, 0)),
            pl.BlockSpec((E, A), lambda i, e: (0, 0)),
        ],
        out_specs=pl.BlockSpec((TILE_T, A), lambda i, e: (i, 0)),
        out_shape=jax.ShapeDtypeStruct((T, A), jnp.float32),
        scratch_shapes=[pltpu.VMEM((TILE_T, E), jnp.float32)],
    )(x, Wg1, bg1, Wg2, bg2, We, be)


def kernel(x, Wg1, bg1, Wg2, bg2, We, be, top_k):
    return _run(x, Wg1, bg1.reshape(1, H), Wg2, bg2.reshape(1, E), We, be)
